# Initial kernel scaffold; baseline (speedup 1.0000x reference)
import jax
import jax.numpy as jnp
from jax.experimental import pallas as pl


def kernel(class_logits, box_regression, proposals):
    raise NotImplementedError("write your pallas kernel here")


# speedup vs baseline: 11.2441x; 11.2441x over previous
import math

import jax
import jax.numpy as jnp
from jax.experimental import pallas as pl

N = 20000
C = 81
NFG = C - 1
IMG_H = 800.0
IMG_W = 1333.0
SCORE_THRESH = 0.05
NMS_THRESH = 0.5
DETS_PER_IMG = 100
MAX_PER_CLASS = 100
NEG = -1e10
BBOX_XFORM_CLIP = math.log(1000.0 / 16.0)
WX, WY, WW, WH = 10.0, 10.0, 5.0, 5.0

LANES = 128
R = 160
NPAD = R * LANES
SM_BLK = 2048


def _softmax_body(lg_ref, out_ref):
    lg = lg_ref[...]
    m = jnp.max(lg, axis=0, keepdims=True)
    e = jnp.exp(lg - m)
    s = jnp.sum(e, axis=0, keepdims=True)
    out_ref[...] = e / s


def _nms_body(prob_ref, breg_ref, prop_ref, osc_ref, obox_ref):
    p = prob_ref[0]
    x1p = prop_ref[0]
    y1p = prop_ref[1]
    x2p = prop_ref[2]
    y2p = prop_ref[3]
    w = x2p - x1p + 1.0
    h = y2p - y1p + 1.0
    cx = x1p + 0.5 * w
    cy = y1p + 0.5 * h
    dx = breg_ref[0, 0] * (1.0 / WX)
    dy = breg_ref[0, 1] * (1.0 / WY)
    dw = jnp.minimum(breg_ref[0, 2] * (1.0 / WW), BBOX_XFORM_CLIP)
    dh = jnp.minimum(breg_ref[0, 3] * (1.0 / WH), BBOX_XFORM_CLIP)
    pcx = dx * w + cx
    pcy = dy * h + cy
    pw = jnp.exp(dw) * w
    ph = jnp.exp(dh) * h
    x1 = jnp.clip(pcx - 0.5 * pw, 0.0, IMG_W - 1.0)
    y1 = jnp.clip(pcy - 0.5 * ph, 0.0, IMG_H - 1.0)
    x2 = jnp.clip(pcx + 0.5 * pw - 1.0, 0.0, IMG_W - 1.0)
    y2 = jnp.clip(pcy + 0.5 * ph - 1.0, 0.0, IMG_H - 1.0)
    areas = (x2 - x1) * (y2 - y1)

    sc0 = jnp.where(p > SCORE_THRESH, p, NEG)
    flat = (jax.lax.broadcasted_iota(jnp.int32, (R, LANES), 0) * LANES
            + jax.lax.broadcasted_iota(jnp.int32, (R, LANES), 1))
    col = jax.lax.broadcasted_iota(jnp.int32, (1, LANES), 1)
    BIG = jnp.int32(2**30)

    def body(i, carry):
        sc, ps, b1, b2, b3, b4 = carry
        m_ = jnp.max(sc)
        eq = sc == m_
        fidx = jnp.min(jnp.where(eq, flat, BIG))
        eqf = flat == fidx
        z = jnp.float32(0.0)
        bx1 = jnp.sum(jnp.where(eqf, x1, z))
        by1 = jnp.sum(jnp.where(eqf, y1, z))
        bx2 = jnp.sum(jnp.where(eqf, x2, z))
        by2 = jnp.sum(jnp.where(eqf, y2, z))
        barea = (bx2 - bx1) * (by2 - by1)
        xx1 = jnp.maximum(x1, bx1)
        yy1 = jnp.maximum(y1, by1)
        xx2 = jnp.minimum(x2, bx2)
        yy2 = jnp.minimum(y2, by2)
        inter = jnp.maximum(xx2 - xx1, 0.0) * jnp.maximum(yy2 - yy1, 0.0)
        iou = inter / (areas + barea - inter + 1e-9)
        sc = jnp.where((iou > NMS_THRESH) | eqf, NEG, sc)
        sel = col == i
        ps = jnp.where(sel, m_, ps)
        b1 = jnp.where(sel, bx1, b1)
        b2 = jnp.where(sel, by1, b2)
        b3 = jnp.where(sel, bx2, b3)
        b4 = jnp.where(sel, by2, b4)
        return sc, ps, b1, b2, b3, b4

    init_v = jnp.full((1, LANES), NEG, jnp.float32)
    zero_v = jnp.zeros((1, LANES), jnp.float32)
    _, ps, b1, b2, b3, b4 = jax.lax.fori_loop(
        0, MAX_PER_CLASS, body, (sc0, init_v, zero_v, zero_v, zero_v, zero_v))
    osc_ref[0] = ps
    obox_ref[0] = jnp.concatenate([b1, b2, b3, b4], axis=0)


def _topk_body(sc_ref, box_ref, fs_ref, fb_ref, fl_ref):
    sc = sc_ref[:, 0, :]
    x1 = box_ref[:, 0, :]
    y1 = box_ref[:, 1, :]
    x2 = box_ref[:, 2, :]
    y2 = box_ref[:, 3, :]
    lab = (jax.lax.broadcasted_iota(jnp.int32, (NFG, LANES), 0) + 1
           ).astype(jnp.float32)
    flat = (jax.lax.broadcasted_iota(jnp.int32, (NFG, LANES), 0) * LANES
            + jax.lax.broadcasted_iota(jnp.int32, (NFG, LANES), 1))
    col = jax.lax.broadcasted_iota(jnp.int32, (1, LANES), 1)
    BIG = jnp.int32(2**30)

    def body(i, carry):
        sc, fs, f1, f2, f3, f4, fl = carry
        m_ = jnp.max(sc)
        eq = sc == m_
        fidx = jnp.min(jnp.where(eq, flat, BIG))
        eqf = flat == fidx
        z = jnp.float32(0.0)
        bx1 = jnp.sum(jnp.where(eqf, x1, z))
        by1 = jnp.sum(jnp.where(eqf, y1, z))
        bx2 = jnp.sum(jnp.where(eqf, x2, z))
        by2 = jnp.sum(jnp.where(eqf, y2, z))
        lb = jnp.sum(jnp.where(eqf, lab, z))
        valid = m_ > SCORE_THRESH
        sv = jnp.where(valid, m_, z)
        b1v = jnp.where(valid, bx1, z)
        b2v = jnp.where(valid, by1, z)
        b3v = jnp.where(valid, bx2, z)
        b4v = jnp.where(valid, by2, z)
        lv = jnp.where(valid, lb, z)
        sel = col == i
        fs = jnp.where(sel, sv, fs)
        f1 = jnp.where(sel, b1v, f1)
        f2 = jnp.where(sel, b2v, f2)
        f3 = jnp.where(sel, b3v, f3)
        f4 = jnp.where(sel, b4v, f4)
        fl = jnp.where(sel, lv, fl)
        sc = jnp.where(eqf, NEG, sc)
        return sc, fs, f1, f2, f3, f4, fl

    zero_v = jnp.zeros((1, LANES), jnp.float32)
    _, fs, f1, f2, f3, f4, fl = jax.lax.fori_loop(
        0, DETS_PER_IMG, body, (sc, zero_v, zero_v, zero_v, zero_v, zero_v, zero_v))
    fs_ref[...] = fs
    fb_ref[...] = jnp.concatenate([f1, f2, f3, f4], axis=0)
    fl_ref[...] = fl


def kernel(class_logits, box_regression, proposals):
    f32 = jnp.float32
    lg_t = jnp.pad(class_logits, ((0, NPAD - N), (0, 0))).T
    br = jnp.pad(box_regression.reshape(N, C, 4), ((0, NPAD - N), (0, 0), (0, 0)))
    br_t = br.transpose(1, 2, 0)[1:].reshape(NFG, 4, R, LANES)
    pr_t = jnp.pad(proposals, ((0, NPAD - N), (0, 0))).T.reshape(4, R, LANES)

    probs = pl.pallas_call(
        _softmax_body,
        grid=(NPAD // SM_BLK,),
        in_specs=[pl.BlockSpec((C, SM_BLK), lambda i: (0, i))],
        out_specs=pl.BlockSpec((C, SM_BLK), lambda i: (0, i)),
        out_shape=jax.ShapeDtypeStruct((C, NPAD), f32),
    )(lg_t)
    probs3 = probs[1:].reshape(NFG, R, LANES)

    nms_sc, nms_box = pl.pallas_call(
        _nms_body,
        grid=(NFG,),
        in_specs=[
            pl.BlockSpec((1, R, LANES), lambda c: (c, 0, 0)),
            pl.BlockSpec((1, 4, R, LANES), lambda c: (c, 0, 0, 0)),
            pl.BlockSpec((4, R, LANES), lambda c: (0, 0, 0)),
        ],
        out_specs=[
            pl.BlockSpec((1, 1, LANES), lambda c: (c, 0, 0)),
            pl.BlockSpec((1, 4, LANES), lambda c: (c, 0, 0)),
        ],
        out_shape=[
            jax.ShapeDtypeStruct((NFG, 1, LANES), f32),
            jax.ShapeDtypeStruct((NFG, 4, LANES), f32),
        ],
    )(probs3, br_t, pr_t)

    fs, fb, fl = pl.pallas_call(
        _topk_body,
        out_shape=[
            jax.ShapeDtypeStruct((1, LANES), f32),
            jax.ShapeDtypeStruct((4, LANES), f32),
            jax.ShapeDtypeStruct((1, LANES), f32),
        ],
    )(nms_sc, nms_box)

    final_scores = fs[0, :DETS_PER_IMG]
    final_boxes = fb[:, :DETS_PER_IMG].T
    final_labels = fl[0, :DETS_PER_IMG].astype(jnp.int32)
    return final_scores, final_boxes, final_labels


# speedup vs baseline: 12.3083x; 1.0947x over previous
import math

import jax
import jax.numpy as jnp
from jax.experimental import pallas as pl

N = 20000
C = 81
NFG = C - 1
IMG_H = 800.0
IMG_W = 1333.0
SCORE_THRESH = 0.05
NMS_THRESH = 0.5
DETS_PER_IMG = 100
MAX_PER_CLASS = 100
NEG = -1e10
BBOX_XFORM_CLIP = math.log(1000.0 / 16.0)
WX, WY, WW, WH = 10.0, 10.0, 5.0, 5.0

LANES = 128
R = 160
NPAD = R * LANES
SM_BLK = 2048


def _softmax_body(lg_ref, out_ref):
    lg = lg_ref[...]
    m = jnp.max(lg, axis=0, keepdims=True)
    e = jnp.exp(lg - m)
    s = jnp.sum(e, axis=0, keepdims=True)
    out_ref[...] = e / s


KPER = 2


def _nms_body(prob_ref, breg_ref, prop_ref, osc_ref, obox_ref):
    x1p = prop_ref[0]
    y1p = prop_ref[1]
    x2p = prop_ref[2]
    y2p = prop_ref[3]
    w = x2p - x1p + 1.0
    h = y2p - y1p + 1.0
    cx = x1p + 0.5 * w
    cy = y1p + 0.5 * h

    x1s, y1s, x2s, y2s, areas_s, sc0s = [], [], [], [], [], []
    for k in range(KPER):
        dx = breg_ref[k, 0] * (1.0 / WX)
        dy = breg_ref[k, 1] * (1.0 / WY)
        dw = jnp.minimum(breg_ref[k, 2] * (1.0 / WW), BBOX_XFORM_CLIP)
        dh = jnp.minimum(breg_ref[k, 3] * (1.0 / WH), BBOX_XFORM_CLIP)
        pcx = dx * w + cx
        pcy = dy * h + cy
        pw = jnp.exp(dw) * w
        ph = jnp.exp(dh) * h
        x1 = jnp.clip(pcx - 0.5 * pw, 0.0, IMG_W - 1.0)
        y1 = jnp.clip(pcy - 0.5 * ph, 0.0, IMG_H - 1.0)
        x2 = jnp.clip(pcx + 0.5 * pw - 1.0, 0.0, IMG_W - 1.0)
        y2 = jnp.clip(pcy + 0.5 * ph - 1.0, 0.0, IMG_H - 1.0)
        x1s.append(x1)
        y1s.append(y1)
        x2s.append(x2)
        y2s.append(y2)
        areas_s.append((x2 - x1) * (y2 - y1))
        p = prob_ref[k]
        sc0s.append(jnp.where(p > SCORE_THRESH, p, NEG))

    flat = (jax.lax.broadcasted_iota(jnp.int32, (R, LANES), 0) * LANES
            + jax.lax.broadcasted_iota(jnp.int32, (R, LANES), 1))
    col = jax.lax.broadcasted_iota(jnp.int32, (1, LANES), 1)
    BIG = jnp.int32(2**30)

    def body(i, carry):
        out = []
        for k in range(KPER):
            sc, ps, b1, b2, b3, b4 = carry[k]
            x1, y1, x2, y2, areas = x1s[k], y1s[k], x2s[k], y2s[k], areas_s[k]
            m_ = jnp.max(sc)
            eq = sc == m_
            fidx = jnp.min(jnp.where(eq, flat, BIG))
            eqf = flat == fidx
            z = jnp.float32(0.0)
            bx1 = jnp.sum(jnp.where(eqf, x1, z))
            by1 = jnp.sum(jnp.where(eqf, y1, z))
            bx2 = jnp.sum(jnp.where(eqf, x2, z))
            by2 = jnp.sum(jnp.where(eqf, y2, z))
            barea = (bx2 - bx1) * (by2 - by1)
            xx1 = jnp.maximum(x1, bx1)
            yy1 = jnp.maximum(y1, by1)
            xx2 = jnp.minimum(x2, bx2)
            yy2 = jnp.minimum(y2, by2)
            inter = jnp.maximum(xx2 - xx1, 0.0) * jnp.maximum(yy2 - yy1, 0.0)
            iou = inter / (areas + barea - inter + 1e-9)
            sc = jnp.where((iou > NMS_THRESH) | eqf, NEG, sc)
            sel = col == i
            ps = jnp.where(sel, m_, ps)
            b1 = jnp.where(sel, bx1, b1)
            b2 = jnp.where(sel, by1, b2)
            b3 = jnp.where(sel, bx2, b3)
            b4 = jnp.where(sel, by2, b4)
            out.append((sc, ps, b1, b2, b3, b4))
        return tuple(out)

    init_v = jnp.full((1, LANES), NEG, jnp.float32)
    zero_v = jnp.zeros((1, LANES), jnp.float32)
    init = tuple((sc0s[k], init_v, zero_v, zero_v, zero_v, zero_v)
                 for k in range(KPER))
    fin = jax.lax.fori_loop(0, MAX_PER_CLASS, body, init)
    for k in range(KPER):
        _, ps, b1, b2, b3, b4 = fin[k]
        osc_ref[k] = ps
        obox_ref[k] = jnp.concatenate([b1, b2, b3, b4], axis=0)


def _topk_body(sc_ref, box_ref, fs_ref, fb_ref, fl_ref):
    sc = sc_ref[:, 0, :]
    x1 = box_ref[:, 0, :]
    y1 = box_ref[:, 1, :]
    x2 = box_ref[:, 2, :]
    y2 = box_ref[:, 3, :]
    lab = (jax.lax.broadcasted_iota(jnp.int32, (NFG, LANES), 0) + 1
           ).astype(jnp.float32)
    flat = (jax.lax.broadcasted_iota(jnp.int32, (NFG, LANES), 0) * LANES
            + jax.lax.broadcasted_iota(jnp.int32, (NFG, LANES), 1))
    col = jax.lax.broadcasted_iota(jnp.int32, (1, LANES), 1)
    BIG = jnp.int32(2**30)

    def body(i, carry):
        sc, fs, f1, f2, f3, f4, fl = carry
        m_ = jnp.max(sc)
        eq = sc == m_
        fidx = jnp.min(jnp.where(eq, flat, BIG))
        eqf = flat == fidx
        z = jnp.float32(0.0)
        bx1 = jnp.sum(jnp.where(eqf, x1, z))
        by1 = jnp.sum(jnp.where(eqf, y1, z))
        bx2 = jnp.sum(jnp.where(eqf, x2, z))
        by2 = jnp.sum(jnp.where(eqf, y2, z))
        lb = jnp.sum(jnp.where(eqf, lab, z))
        valid = m_ > SCORE_THRESH
        sv = jnp.where(valid, m_, z)
        b1v = jnp.where(valid, bx1, z)
        b2v = jnp.where(valid, by1, z)
        b3v = jnp.where(valid, bx2, z)
        b4v = jnp.where(valid, by2, z)
        lv = jnp.where(valid, lb, z)
        sel = col == i
        fs = jnp.where(sel, sv, fs)
        f1 = jnp.where(sel, b1v, f1)
        f2 = jnp.where(sel, b2v, f2)
        f3 = jnp.where(sel, b3v, f3)
        f4 = jnp.where(sel, b4v, f4)
        fl = jnp.where(sel, lv, fl)
        sc = jnp.where(eqf, NEG, sc)
        return sc, fs, f1, f2, f3, f4, fl

    zero_v = jnp.zeros((1, LANES), jnp.float32)
    _, fs, f1, f2, f3, f4, fl = jax.lax.fori_loop(
        0, DETS_PER_IMG, body, (sc, zero_v, zero_v, zero_v, zero_v, zero_v, zero_v))
    fs_ref[...] = fs
    fb_ref[...] = jnp.concatenate([f1, f2, f3, f4], axis=0)
    fl_ref[...] = fl


def kernel(class_logits, box_regression, proposals):
    f32 = jnp.float32
    lg_t = jnp.pad(class_logits, ((0, NPAD - N), (0, 0))).T
    br = jnp.pad(box_regression.reshape(N, C, 4), ((0, NPAD - N), (0, 0), (0, 0)))
    br_t = br.transpose(1, 2, 0)[1:].reshape(NFG, 4, R, LANES)
    pr_t = jnp.pad(proposals, ((0, NPAD - N), (0, 0))).T.reshape(4, R, LANES)

    probs = pl.pallas_call(
        _softmax_body,
        grid=(NPAD // SM_BLK,),
        in_specs=[pl.BlockSpec((C, SM_BLK), lambda i: (0, i))],
        out_specs=pl.BlockSpec((C, SM_BLK), lambda i: (0, i)),
        out_shape=jax.ShapeDtypeStruct((C, NPAD), f32),
    )(lg_t)
    probs3 = probs[1:].reshape(NFG, R, LANES)

    nms_sc, nms_box = pl.pallas_call(
        _nms_body,
        grid=(NFG // KPER,),
        in_specs=[
            pl.BlockSpec((KPER, R, LANES), lambda c: (c, 0, 0)),
            pl.BlockSpec((KPER, 4, R, LANES), lambda c: (c, 0, 0, 0)),
            pl.BlockSpec((4, R, LANES), lambda c: (0, 0, 0)),
        ],
        out_specs=[
            pl.BlockSpec((KPER, 1, LANES), lambda c: (c, 0, 0)),
            pl.BlockSpec((KPER, 4, LANES), lambda c: (c, 0, 0)),
        ],
        out_shape=[
            jax.ShapeDtypeStruct((NFG, 1, LANES), f32),
            jax.ShapeDtypeStruct((NFG, 4, LANES), f32),
        ],
    )(probs3, br_t, pr_t)

    fs, fb, fl = pl.pallas_call(
        _topk_body,
        out_shape=[
            jax.ShapeDtypeStruct((1, LANES), f32),
            jax.ShapeDtypeStruct((4, LANES), f32),
            jax.ShapeDtypeStruct((1, LANES), f32),
        ],
    )(nms_sc, nms_box)

    final_scores = fs[0, :DETS_PER_IMG]
    final_boxes = fb[:, :DETS_PER_IMG].T
    final_labels = fl[0, :DETS_PER_IMG].astype(jnp.int32)
    return final_scores, final_boxes, final_labels


# speedup vs baseline: 12.8726x; 1.0458x over previous
import math

import jax
import jax.numpy as jnp
from jax.experimental import pallas as pl

N = 20000
C = 81
NFG = C - 1
IMG_H = 800.0
IMG_W = 1333.0
SCORE_THRESH = 0.05
NMS_THRESH = 0.5
DETS_PER_IMG = 100
MAX_PER_CLASS = 100
NEG = -1e10
BBOX_XFORM_CLIP = math.log(1000.0 / 16.0)
WX, WY, WW, WH = 10.0, 10.0, 5.0, 5.0

LANES = 128
R = 160
NPAD = R * LANES
SM_BLK = 2048


def _softmax_body(lg_ref, out_ref):
    lg = lg_ref[...]
    m = jnp.max(lg, axis=0, keepdims=True)
    e = jnp.exp(lg - m)
    s = jnp.sum(e, axis=0, keepdims=True)
    out_ref[...] = e / s


KPER = 4


def _nms_body(prob_ref, breg_ref, prop_ref, osc_ref, obox_ref):
    x1p = prop_ref[0]
    y1p = prop_ref[1]
    x2p = prop_ref[2]
    y2p = prop_ref[3]
    w = x2p - x1p + 1.0
    h = y2p - y1p + 1.0
    cx = x1p + 0.5 * w
    cy = y1p + 0.5 * h

    x1s, y1s, x2s, y2s, areas_s, sc0s = [], [], [], [], [], []
    for k in range(KPER):
        dx = breg_ref[k, 0] * (1.0 / WX)
        dy = breg_ref[k, 1] * (1.0 / WY)
        dw = jnp.minimum(breg_ref[k, 2] * (1.0 / WW), BBOX_XFORM_CLIP)
        dh = jnp.minimum(breg_ref[k, 3] * (1.0 / WH), BBOX_XFORM_CLIP)
        pcx = dx * w + cx
        pcy = dy * h + cy
        pw = jnp.exp(dw) * w
        ph = jnp.exp(dh) * h
        x1 = jnp.clip(pcx - 0.5 * pw, 0.0, IMG_W - 1.0)
        y1 = jnp.clip(pcy - 0.5 * ph, 0.0, IMG_H - 1.0)
        x2 = jnp.clip(pcx + 0.5 * pw - 1.0, 0.0, IMG_W - 1.0)
        y2 = jnp.clip(pcy + 0.5 * ph - 1.0, 0.0, IMG_H - 1.0)
        x1s.append(x1)
        y1s.append(y1)
        x2s.append(x2)
        y2s.append(y2)
        areas_s.append((x2 - x1) * (y2 - y1))
        p = prob_ref[k]
        sc0s.append(jnp.where(p > SCORE_THRESH, p, NEG))

    flat = (jax.lax.broadcasted_iota(jnp.int32, (R, LANES), 0) * LANES
            + jax.lax.broadcasted_iota(jnp.int32, (R, LANES), 1))
    col = jax.lax.broadcasted_iota(jnp.int32, (1, LANES), 1)
    BIG = jnp.int32(2**30)

    def body(i, carry):
        out = []
        for k in range(KPER):
            sc, ps, b1, b2, b3, b4 = carry[k]
            x1, y1, x2, y2, areas = x1s[k], y1s[k], x2s[k], y2s[k], areas_s[k]
            m_ = jnp.max(sc)
            eq = sc == m_
            fidx = jnp.min(jnp.where(eq, flat, BIG))
            eqf = flat == fidx
            z = jnp.float32(0.0)
            bx1 = jnp.sum(jnp.where(eqf, x1, z))
            by1 = jnp.sum(jnp.where(eqf, y1, z))
            bx2 = jnp.sum(jnp.where(eqf, x2, z))
            by2 = jnp.sum(jnp.where(eqf, y2, z))
            barea = (bx2 - bx1) * (by2 - by1)
            xx1 = jnp.maximum(x1, bx1)
            yy1 = jnp.maximum(y1, by1)
            xx2 = jnp.minimum(x2, bx2)
            yy2 = jnp.minimum(y2, by2)
            inter = jnp.maximum(xx2 - xx1, 0.0) * jnp.maximum(yy2 - yy1, 0.0)
            iou = inter / (areas + barea - inter + 1e-9)
            sc = jnp.where((iou > NMS_THRESH) | eqf, NEG, sc)
            sel = col == i
            ps = jnp.where(sel, m_, ps)
            b1 = jnp.where(sel, bx1, b1)
            b2 = jnp.where(sel, by1, b2)
            b3 = jnp.where(sel, bx2, b3)
            b4 = jnp.where(sel, by2, b4)
            out.append((sc, ps, b1, b2, b3, b4))
        return tuple(out)

    init_v = jnp.full((1, LANES), NEG, jnp.float32)
    zero_v = jnp.zeros((1, LANES), jnp.float32)
    init = tuple((sc0s[k], init_v, zero_v, zero_v, zero_v, zero_v)
                 for k in range(KPER))
    fin = jax.lax.fori_loop(0, MAX_PER_CLASS, body, init)
    for k in range(KPER):
        _, ps, b1, b2, b3, b4 = fin[k]
        osc_ref[k] = ps
        obox_ref[k] = jnp.concatenate([b1, b2, b3, b4], axis=0)


def _topk_body(sc_ref, box_ref, fs_ref, fb_ref, fl_ref):
    sc = sc_ref[:, 0, :]
    x1 = box_ref[:, 0, :]
    y1 = box_ref[:, 1, :]
    x2 = box_ref[:, 2, :]
    y2 = box_ref[:, 3, :]
    lab = (jax.lax.broadcasted_iota(jnp.int32, (NFG, LANES), 0) + 1
           ).astype(jnp.float32)
    flat = (jax.lax.broadcasted_iota(jnp.int32, (NFG, LANES), 0) * LANES
            + jax.lax.broadcasted_iota(jnp.int32, (NFG, LANES), 1))
    col = jax.lax.broadcasted_iota(jnp.int32, (1, LANES), 1)
    BIG = jnp.int32(2**30)

    def body(i, carry):
        sc, fs, f1, f2, f3, f4, fl = carry
        m_ = jnp.max(sc)
        eq = sc == m_
        fidx = jnp.min(jnp.where(eq, flat, BIG))
        eqf = flat == fidx
        z = jnp.float32(0.0)
        bx1 = jnp.sum(jnp.where(eqf, x1, z))
        by1 = jnp.sum(jnp.where(eqf, y1, z))
        bx2 = jnp.sum(jnp.where(eqf, x2, z))
        by2 = jnp.sum(jnp.where(eqf, y2, z))
        lb = jnp.sum(jnp.where(eqf, lab, z))
        valid = m_ > SCORE_THRESH
        sv = jnp.where(valid, m_, z)
        b1v = jnp.where(valid, bx1, z)
        b2v = jnp.where(valid, by1, z)
        b3v = jnp.where(valid, bx2, z)
        b4v = jnp.where(valid, by2, z)
        lv = jnp.where(valid, lb, z)
        sel = col == i
        fs = jnp.where(sel, sv, fs)
        f1 = jnp.where(sel, b1v, f1)
        f2 = jnp.where(sel, b2v, f2)
        f3 = jnp.where(sel, b3v, f3)
        f4 = jnp.where(sel, b4v, f4)
        fl = jnp.where(sel, lv, fl)
        sc = jnp.where(eqf, NEG, sc)
        return sc, fs, f1, f2, f3, f4, fl

    zero_v = jnp.zeros((1, LANES), jnp.float32)
    _, fs, f1, f2, f3, f4, fl = jax.lax.fori_loop(
        0, DETS_PER_IMG, body, (sc, zero_v, zero_v, zero_v, zero_v, zero_v, zero_v))
    fs_ref[...] = fs
    fb_ref[...] = jnp.concatenate([f1, f2, f3, f4], axis=0)
    fl_ref[...] = fl


def kernel(class_logits, box_regression, proposals):
    f32 = jnp.float32
    lg_t = jnp.pad(class_logits, ((0, NPAD - N), (0, 0))).T
    br = jnp.pad(box_regression.reshape(N, C, 4), ((0, NPAD - N), (0, 0), (0, 0)))
    br_t = br.transpose(1, 2, 0)[1:].reshape(NFG, 4, R, LANES)
    pr_t = jnp.pad(proposals, ((0, NPAD - N), (0, 0))).T.reshape(4, R, LANES)

    probs = pl.pallas_call(
        _softmax_body,
        grid=(NPAD // SM_BLK,),
        in_specs=[pl.BlockSpec((C, SM_BLK), lambda i: (0, i))],
        out_specs=pl.BlockSpec((C, SM_BLK), lambda i: (0, i)),
        out_shape=jax.ShapeDtypeStruct((C, NPAD), f32),
    )(lg_t)
    probs3 = probs[1:].reshape(NFG, R, LANES)

    nms_sc, nms_box = pl.pallas_call(
        _nms_body,
        grid=(NFG // KPER,),
        in_specs=[
            pl.BlockSpec((KPER, R, LANES), lambda c: (c, 0, 0)),
            pl.BlockSpec((KPER, 4, R, LANES), lambda c: (c, 0, 0, 0)),
            pl.BlockSpec((4, R, LANES), lambda c: (0, 0, 0)),
        ],
        out_specs=[
            pl.BlockSpec((KPER, 1, LANES), lambda c: (c, 0, 0)),
            pl.BlockSpec((KPER, 4, LANES), lambda c: (c, 0, 0)),
        ],
        out_shape=[
            jax.ShapeDtypeStruct((NFG, 1, LANES), f32),
            jax.ShapeDtypeStruct((NFG, 4, LANES), f32),
        ],
    )(probs3, br_t, pr_t)

    fs, fb, fl = pl.pallas_call(
        _topk_body,
        out_shape=[
            jax.ShapeDtypeStruct((1, LANES), f32),
            jax.ShapeDtypeStruct((4, LANES), f32),
            jax.ShapeDtypeStruct((1, LANES), f32),
        ],
    )(nms_sc, nms_box)

    final_scores = fs[0, :DETS_PER_IMG]
    final_boxes = fb[:, :DETS_PER_IMG].T
    final_labels = fl[0, :DETS_PER_IMG].astype(jnp.int32)
    return final_scores, final_boxes, final_labels


# speedup vs baseline: 13.2837x; 1.0319x over previous
import math

import jax
import jax.numpy as jnp
from jax.experimental import pallas as pl

N = 20000
C = 81
NFG = C - 1
IMG_H = 800.0
IMG_W = 1333.0
SCORE_THRESH = 0.05
NMS_THRESH = 0.5
DETS_PER_IMG = 100
MAX_PER_CLASS = 100
NEG = -1e10
BBOX_XFORM_CLIP = math.log(1000.0 / 16.0)
WX, WY, WW, WH = 10.0, 10.0, 5.0, 5.0

LANES = 128
R = 160
NPAD = R * LANES
SM_BLK = 2048


def _softmax_body(lg_ref, out_ref):
    lg = lg_ref[...]
    m = jnp.max(lg, axis=0, keepdims=True)
    e = jnp.exp(lg - m)
    s = jnp.sum(e, axis=0, keepdims=True)
    out_ref[...] = e / s


KPER = 8


def _nms_body(prob_ref, breg_ref, prop_ref, osc_ref, obox_ref):
    x1p = prop_ref[0]
    y1p = prop_ref[1]
    x2p = prop_ref[2]
    y2p = prop_ref[3]
    w = x2p - x1p + 1.0
    h = y2p - y1p + 1.0
    cx = x1p + 0.5 * w
    cy = y1p + 0.5 * h

    x1s, y1s, x2s, y2s, areas_s, sc0s = [], [], [], [], [], []
    for k in range(KPER):
        dx = breg_ref[k, 0] * (1.0 / WX)
        dy = breg_ref[k, 1] * (1.0 / WY)
        dw = jnp.minimum(breg_ref[k, 2] * (1.0 / WW), BBOX_XFORM_CLIP)
        dh = jnp.minimum(breg_ref[k, 3] * (1.0 / WH), BBOX_XFORM_CLIP)
        pcx = dx * w + cx
        pcy = dy * h + cy
        pw = jnp.exp(dw) * w
        ph = jnp.exp(dh) * h
        x1 = jnp.clip(pcx - 0.5 * pw, 0.0, IMG_W - 1.0)
        y1 = jnp.clip(pcy - 0.5 * ph, 0.0, IMG_H - 1.0)
        x2 = jnp.clip(pcx + 0.5 * pw - 1.0, 0.0, IMG_W - 1.0)
        y2 = jnp.clip(pcy + 0.5 * ph - 1.0, 0.0, IMG_H - 1.0)
        x1s.append(x1)
        y1s.append(y1)
        x2s.append(x2)
        y2s.append(y2)
        areas_s.append((x2 - x1) * (y2 - y1))
        p = prob_ref[k]
        sc0s.append(jnp.where(p > SCORE_THRESH, p, NEG))

    flat = (jax.lax.broadcasted_iota(jnp.int32, (R, LANES), 0) * LANES
            + jax.lax.broadcasted_iota(jnp.int32, (R, LANES), 1))
    col = jax.lax.broadcasted_iota(jnp.int32, (1, LANES), 1)
    BIG = jnp.int32(2**30)

    def body(i, carry):
        out = []
        for k in range(KPER):
            sc, ps, b1, b2, b3, b4 = carry[k]
            x1, y1, x2, y2, areas = x1s[k], y1s[k], x2s[k], y2s[k], areas_s[k]
            m_ = jnp.max(sc)
            eq = sc == m_
            fidx = jnp.min(jnp.where(eq, flat, BIG))
            eqf = flat == fidx
            z = jnp.float32(0.0)
            bx1 = jnp.sum(jnp.where(eqf, x1, z))
            by1 = jnp.sum(jnp.where(eqf, y1, z))
            bx2 = jnp.sum(jnp.where(eqf, x2, z))
            by2 = jnp.sum(jnp.where(eqf, y2, z))
            barea = (bx2 - bx1) * (by2 - by1)
            xx1 = jnp.maximum(x1, bx1)
            yy1 = jnp.maximum(y1, by1)
            xx2 = jnp.minimum(x2, bx2)
            yy2 = jnp.minimum(y2, by2)
            inter = jnp.maximum(xx2 - xx1, 0.0) * jnp.maximum(yy2 - yy1, 0.0)
            iou = inter / (areas + barea - inter + 1e-9)
            sc = jnp.where((iou > NMS_THRESH) | eqf, NEG, sc)
            sel = col == i
            ps = jnp.where(sel, m_, ps)
            b1 = jnp.where(sel, bx1, b1)
            b2 = jnp.where(sel, by1, b2)
            b3 = jnp.where(sel, bx2, b3)
            b4 = jnp.where(sel, by2, b4)
            out.append((sc, ps, b1, b2, b3, b4))
        return tuple(out)

    init_v = jnp.full((1, LANES), NEG, jnp.float32)
    zero_v = jnp.zeros((1, LANES), jnp.float32)
    init = tuple((sc0s[k], init_v, zero_v, zero_v, zero_v, zero_v)
                 for k in range(KPER))
    fin = jax.lax.fori_loop(0, MAX_PER_CLASS, body, init)
    for k in range(KPER):
        _, ps, b1, b2, b3, b4 = fin[k]
        osc_ref[k] = ps
        obox_ref[k] = jnp.concatenate([b1, b2, b3, b4], axis=0)


def _topk_body(sc_ref, box_ref, fs_ref, fb_ref, fl_ref):
    sc = sc_ref[:, 0, :]
    x1 = box_ref[:, 0, :]
    y1 = box_ref[:, 1, :]
    x2 = box_ref[:, 2, :]
    y2 = box_ref[:, 3, :]
    lab = (jax.lax.broadcasted_iota(jnp.int32, (NFG, LANES), 0) + 1
           ).astype(jnp.float32)
    flat = (jax.lax.broadcasted_iota(jnp.int32, (NFG, LANES), 0) * LANES
            + jax.lax.broadcasted_iota(jnp.int32, (NFG, LANES), 1))
    col = jax.lax.broadcasted_iota(jnp.int32, (1, LANES), 1)
    BIG = jnp.int32(2**30)

    def body(i, carry):
        sc, fs, f1, f2, f3, f4, fl = carry
        m_ = jnp.max(sc)
        eq = sc == m_
        fidx = jnp.min(jnp.where(eq, flat, BIG))
        eqf = flat == fidx
        z = jnp.float32(0.0)
        bx1 = jnp.sum(jnp.where(eqf, x1, z))
        by1 = jnp.sum(jnp.where(eqf, y1, z))
        bx2 = jnp.sum(jnp.where(eqf, x2, z))
        by2 = jnp.sum(jnp.where(eqf, y2, z))
        lb = jnp.sum(jnp.where(eqf, lab, z))
        valid = m_ > SCORE_THRESH
        sv = jnp.where(valid, m_, z)
        b1v = jnp.where(valid, bx1, z)
        b2v = jnp.where(valid, by1, z)
        b3v = jnp.where(valid, bx2, z)
        b4v = jnp.where(valid, by2, z)
        lv = jnp.where(valid, lb, z)
        sel = col == i
        fs = jnp.where(sel, sv, fs)
        f1 = jnp.where(sel, b1v, f1)
        f2 = jnp.where(sel, b2v, f2)
        f3 = jnp.where(sel, b3v, f3)
        f4 = jnp.where(sel, b4v, f4)
        fl = jnp.where(sel, lv, fl)
        sc = jnp.where(eqf, NEG, sc)
        return sc, fs, f1, f2, f3, f4, fl

    zero_v = jnp.zeros((1, LANES), jnp.float32)
    _, fs, f1, f2, f3, f4, fl = jax.lax.fori_loop(
        0, DETS_PER_IMG, body, (sc, zero_v, zero_v, zero_v, zero_v, zero_v, zero_v))
    fs_ref[...] = fs
    fb_ref[...] = jnp.concatenate([f1, f2, f3, f4], axis=0)
    fl_ref[...] = fl


def kernel(class_logits, box_regression, proposals):
    f32 = jnp.float32
    lg_t = jnp.pad(class_logits, ((0, NPAD - N), (0, 0))).T
    br = jnp.pad(box_regression.reshape(N, C, 4), ((0, NPAD - N), (0, 0), (0, 0)))
    br_t = br.transpose(1, 2, 0)[1:].reshape(NFG, 4, R, LANES)
    pr_t = jnp.pad(proposals, ((0, NPAD - N), (0, 0))).T.reshape(4, R, LANES)

    probs = pl.pallas_call(
        _softmax_body,
        grid=(NPAD // SM_BLK,),
        in_specs=[pl.BlockSpec((C, SM_BLK), lambda i: (0, i))],
        out_specs=pl.BlockSpec((C, SM_BLK), lambda i: (0, i)),
        out_shape=jax.ShapeDtypeStruct((C, NPAD), f32),
    )(lg_t)
    probs3 = probs[1:].reshape(NFG, R, LANES)

    nms_sc, nms_box = pl.pallas_call(
        _nms_body,
        grid=(NFG // KPER,),
        in_specs=[
            pl.BlockSpec((KPER, R, LANES), lambda c: (c, 0, 0)),
            pl.BlockSpec((KPER, 4, R, LANES), lambda c: (c, 0, 0, 0)),
            pl.BlockSpec((4, R, LANES), lambda c: (0, 0, 0)),
        ],
        out_specs=[
            pl.BlockSpec((KPER, 1, LANES), lambda c: (c, 0, 0)),
            pl.BlockSpec((KPER, 4, LANES), lambda c: (c, 0, 0)),
        ],
        out_shape=[
            jax.ShapeDtypeStruct((NFG, 1, LANES), f32),
            jax.ShapeDtypeStruct((NFG, 4, LANES), f32),
        ],
    )(probs3, br_t, pr_t)

    fs, fb, fl = pl.pallas_call(
        _topk_body,
        out_shape=[
            jax.ShapeDtypeStruct((1, LANES), f32),
            jax.ShapeDtypeStruct((4, LANES), f32),
            jax.ShapeDtypeStruct((1, LANES), f32),
        ],
    )(nms_sc, nms_box)

    final_scores = fs[0, :DETS_PER_IMG]
    final_boxes = fb[:, :DETS_PER_IMG].T
    final_labels = fl[0, :DETS_PER_IMG].astype(jnp.int32)
    return final_scores, final_boxes, final_labels


# speedup vs baseline: 13.4118x; 1.0096x over previous
import math

import jax
import jax.numpy as jnp
from jax.experimental import pallas as pl

N = 20000
C = 81
NFG = C - 1
IMG_H = 800.0
IMG_W = 1333.0
SCORE_THRESH = 0.05
NMS_THRESH = 0.5
DETS_PER_IMG = 100
MAX_PER_CLASS = 100
NEG = -1e10
BBOX_XFORM_CLIP = math.log(1000.0 / 16.0)
WX, WY, WW, WH = 10.0, 10.0, 5.0, 5.0

LANES = 128
R = 160
NPAD = R * LANES
SM_BLK = 2048


def _softmax_body(lg_ref, out_ref):
    lg = lg_ref[...]
    m = jnp.max(lg, axis=0, keepdims=True)
    e = jnp.exp(lg - m)
    s = jnp.sum(e, axis=0, keepdims=True)
    out_ref[...] = e / s


KPER = 10


def _nms_body(prob_ref, breg_ref, prop_ref, osc_ref, obox_ref):
    x1p = prop_ref[0]
    y1p = prop_ref[1]
    x2p = prop_ref[2]
    y2p = prop_ref[3]
    w = x2p - x1p + 1.0
    h = y2p - y1p + 1.0
    cx = x1p + 0.5 * w
    cy = y1p + 0.5 * h

    x1s, y1s, x2s, y2s, areas_s, sc0s = [], [], [], [], [], []
    for k in range(KPER):
        dx = breg_ref[k, 0] * (1.0 / WX)
        dy = breg_ref[k, 1] * (1.0 / WY)
        dw = jnp.minimum(breg_ref[k, 2] * (1.0 / WW), BBOX_XFORM_CLIP)
        dh = jnp.minimum(breg_ref[k, 3] * (1.0 / WH), BBOX_XFORM_CLIP)
        pcx = dx * w + cx
        pcy = dy * h + cy
        pw = jnp.exp(dw) * w
        ph = jnp.exp(dh) * h
        x1 = jnp.clip(pcx - 0.5 * pw, 0.0, IMG_W - 1.0)
        y1 = jnp.clip(pcy - 0.5 * ph, 0.0, IMG_H - 1.0)
        x2 = jnp.clip(pcx + 0.5 * pw - 1.0, 0.0, IMG_W - 1.0)
        y2 = jnp.clip(pcy + 0.5 * ph - 1.0, 0.0, IMG_H - 1.0)
        x1s.append(x1)
        y1s.append(y1)
        x2s.append(x2)
        y2s.append(y2)
        areas_s.append((x2 - x1) * (y2 - y1))
        p = prob_ref[k]
        sc0s.append(jnp.where(p > SCORE_THRESH, p, NEG))

    flat = (jax.lax.broadcasted_iota(jnp.int32, (R, LANES), 0) * LANES
            + jax.lax.broadcasted_iota(jnp.int32, (R, LANES), 1))
    col = jax.lax.broadcasted_iota(jnp.int32, (1, LANES), 1)
    BIG = jnp.int32(2**30)

    def body(i, carry):
        out = []
        for k in range(KPER):
            sc, ps, b1, b2, b3, b4 = carry[k]
            x1, y1, x2, y2, areas = x1s[k], y1s[k], x2s[k], y2s[k], areas_s[k]
            m_ = jnp.max(sc)
            eq = sc == m_
            fidx = jnp.min(jnp.where(eq, flat, BIG))
            eqf = flat == fidx
            z = jnp.float32(0.0)
            bx1 = jnp.sum(jnp.where(eqf, x1, z))
            by1 = jnp.sum(jnp.where(eqf, y1, z))
            bx2 = jnp.sum(jnp.where(eqf, x2, z))
            by2 = jnp.sum(jnp.where(eqf, y2, z))
            barea = (bx2 - bx1) * (by2 - by1)
            xx1 = jnp.maximum(x1, bx1)
            yy1 = jnp.maximum(y1, by1)
            xx2 = jnp.minimum(x2, bx2)
            yy2 = jnp.minimum(y2, by2)
            inter = jnp.maximum(xx2 - xx1, 0.0) * jnp.maximum(yy2 - yy1, 0.0)
            iou = inter / (areas + barea - inter + 1e-9)
            sc = jnp.where((iou > NMS_THRESH) | eqf, NEG, sc)
            sel = col == i
            ps = jnp.where(sel, m_, ps)
            b1 = jnp.where(sel, bx1, b1)
            b2 = jnp.where(sel, by1, b2)
            b3 = jnp.where(sel, bx2, b3)
            b4 = jnp.where(sel, by2, b4)
            out.append((sc, ps, b1, b2, b3, b4))
        return tuple(out)

    init_v = jnp.full((1, LANES), NEG, jnp.float32)
    zero_v = jnp.zeros((1, LANES), jnp.float32)
    init = tuple((sc0s[k], init_v, zero_v, zero_v, zero_v, zero_v)
                 for k in range(KPER))
    fin = jax.lax.fori_loop(0, MAX_PER_CLASS, body, init)
    for k in range(KPER):
        _, ps, b1, b2, b3, b4 = fin[k]
        osc_ref[k] = ps
        obox_ref[k] = jnp.concatenate([b1, b2, b3, b4], axis=0)


def _topk_body(sc_ref, box_ref, fs_ref, fb_ref, fl_ref):
    sc = sc_ref[:, 0, :]
    x1 = box_ref[:, 0, :]
    y1 = box_ref[:, 1, :]
    x2 = box_ref[:, 2, :]
    y2 = box_ref[:, 3, :]
    lab = (jax.lax.broadcasted_iota(jnp.int32, (NFG, LANES), 0) + 1
           ).astype(jnp.float32)
    flat = (jax.lax.broadcasted_iota(jnp.int32, (NFG, LANES), 0) * LANES
            + jax.lax.broadcasted_iota(jnp.int32, (NFG, LANES), 1))
    col = jax.lax.broadcasted_iota(jnp.int32, (1, LANES), 1)
    BIG = jnp.int32(2**30)

    def body(i, carry):
        sc, fs, f1, f2, f3, f4, fl = carry
        m_ = jnp.max(sc)
        eq = sc == m_
        fidx = jnp.min(jnp.where(eq, flat, BIG))
        eqf = flat == fidx
        z = jnp.float32(0.0)
        bx1 = jnp.sum(jnp.where(eqf, x1, z))
        by1 = jnp.sum(jnp.where(eqf, y1, z))
        bx2 = jnp.sum(jnp.where(eqf, x2, z))
        by2 = jnp.sum(jnp.where(eqf, y2, z))
        lb = jnp.sum(jnp.where(eqf, lab, z))
        valid = m_ > SCORE_THRESH
        sv = jnp.where(valid, m_, z)
        b1v = jnp.where(valid, bx1, z)
        b2v = jnp.where(valid, by1, z)
        b3v = jnp.where(valid, bx2, z)
        b4v = jnp.where(valid, by2, z)
        lv = jnp.where(valid, lb, z)
        sel = col == i
        fs = jnp.where(sel, sv, fs)
        f1 = jnp.where(sel, b1v, f1)
        f2 = jnp.where(sel, b2v, f2)
        f3 = jnp.where(sel, b3v, f3)
        f4 = jnp.where(sel, b4v, f4)
        fl = jnp.where(sel, lv, fl)
        sc = jnp.where(eqf, NEG, sc)
        return sc, fs, f1, f2, f3, f4, fl

    zero_v = jnp.zeros((1, LANES), jnp.float32)
    _, fs, f1, f2, f3, f4, fl = jax.lax.fori_loop(
        0, DETS_PER_IMG, body, (sc, zero_v, zero_v, zero_v, zero_v, zero_v, zero_v))
    fs_ref[...] = fs
    fb_ref[...] = jnp.concatenate([f1, f2, f3, f4], axis=0)
    fl_ref[...] = fl


def kernel(class_logits, box_regression, proposals):
    f32 = jnp.float32
    lg_t = jnp.pad(class_logits, ((0, NPAD - N), (0, 0))).T
    br = jnp.pad(box_regression.reshape(N, C, 4), ((0, NPAD - N), (0, 0), (0, 0)))
    br_t = br.transpose(1, 2, 0)[1:].reshape(NFG, 4, R, LANES)
    pr_t = jnp.pad(proposals, ((0, NPAD - N), (0, 0))).T.reshape(4, R, LANES)

    probs = pl.pallas_call(
        _softmax_body,
        grid=(NPAD // SM_BLK,),
        in_specs=[pl.BlockSpec((C, SM_BLK), lambda i: (0, i))],
        out_specs=pl.BlockSpec((C, SM_BLK), lambda i: (0, i)),
        out_shape=jax.ShapeDtypeStruct((C, NPAD), f32),
    )(lg_t)
    probs3 = probs[1:].reshape(NFG, R, LANES)

    nms_sc, nms_box = pl.pallas_call(
        _nms_body,
        grid=(NFG // KPER,),
        in_specs=[
            pl.BlockSpec((KPER, R, LANES), lambda c: (c, 0, 0)),
            pl.BlockSpec((KPER, 4, R, LANES), lambda c: (c, 0, 0, 0)),
            pl.BlockSpec((4, R, LANES), lambda c: (0, 0, 0)),
        ],
        out_specs=[
            pl.BlockSpec((KPER, 1, LANES), lambda c: (c, 0, 0)),
            pl.BlockSpec((KPER, 4, LANES), lambda c: (c, 0, 0)),
        ],
        out_shape=[
            jax.ShapeDtypeStruct((NFG, 1, LANES), f32),
            jax.ShapeDtypeStruct((NFG, 4, LANES), f32),
        ],
    )(probs3, br_t, pr_t)

    fs, fb, fl = pl.pallas_call(
        _topk_body,
        out_shape=[
            jax.ShapeDtypeStruct((1, LANES), f32),
            jax.ShapeDtypeStruct((4, LANES), f32),
            jax.ShapeDtypeStruct((1, LANES), f32),
        ],
    )(nms_sc, nms_box)

    final_scores = fs[0, :DETS_PER_IMG]
    final_boxes = fb[:, :DETS_PER_IMG].T
    final_labels = fl[0, :DETS_PER_IMG].astype(jnp.int32)
    return final_scores, final_boxes, final_labels
